# baseline (speedup 1.0000x reference)
import functools

import jax
import jax.numpy as jnp
from jax import lax
from jax.experimental import pallas as pl
from jax.experimental.pallas import tpu as pltpu
from jax.experimental.pallas import tpu_sc as plsc

N = 100000
D = 128
K = 15
NUM_CLASSES = 10

B = 5000
NH = N // 2
NBH = NH // B

NC = 2
NS = 16
L = 16
CH = 3136
CHS = NH - (NS - 1) * CH
HCH = CH // 2
NREF = 32


def _dist_body(x_ref, t_ref, o_ref):
    t = t_ref[...]
    diff = t - x_ref[...]
    sq = (diff * diff).astype(jnp.bfloat16)
    ones = jnp.ones((1, D), jnp.bfloat16)
    d = lax.dot_general(ones, sq, (((1,), (1,)), ((), ())),
                        preferred_element_type=jnp.float32)
    o_ref[...] = d.reshape(1, 1, B)


def _make_dist_half(off):
    return pl.pallas_call(
        _dist_body,
        grid=(NBH,),
        in_specs=[
            pl.BlockSpec((1, D), lambda i: (0, 0)),
            pl.BlockSpec((B, D), lambda i: (i + off, 0)),
        ],
        out_specs=pl.BlockSpec((1, 1, B), lambda i: (i, 0, 0)),
        out_shape=jax.ShapeDtypeStruct((NBH, 1, B), jnp.float32),
    )


_dist_half0 = _make_dist_half(0)
_dist_half1 = _make_dist_half(NBH)


def _merge16(bk, bv, ck, cv):
    ckr = lax.rev(ck, (0,))
    cvr = lax.rev(cv, (0,))
    m = ckr < bk
    mk = jnp.where(m, ckr, bk)
    mv = jnp.where(m, cvr, bv)
    mk2, mv2 = plsc.sort_key_val(mk, mv)
    return mk2, mv2


def _merge16_both(bk, bv, ck, cv):
    ckr = lax.rev(ck, (0,))
    cvr = lax.rev(cv, (0,))
    m = ckr < bk
    lk = jnp.where(m, ckr, bk)
    lv = jnp.where(m, cvr, bv)
    uk = jnp.where(m, bk, ckr)
    uv = jnp.where(m, bv, cvr)
    lk2, lv2 = plsc.sort_key_val(lk, lv)
    uk2, uv2 = plsc.sort_key_val(uk, uv)
    return lk2, lv2, uk2, uv2


def _scan_chunk(dist_hbm, dbuf, s, gbase):
    base = s * CH
    inf16 = jnp.full((L,), jnp.inf, jnp.float32)
    for t in range((CH - CHS) // L):
        dbuf[pl.ds(CHS + t * L, L)] = inf16

    @pl.when(s < NS - 1)
    def _():
        pltpu.sync_copy(dist_hbm.at[pl.ds(base, CH)], dbuf)

    @pl.when(s == NS - 1)
    def _():
        pltpu.sync_copy(dist_hbm.at[pl.ds(base, CHS)],
                        dbuf.at[pl.ds(0, CHS)])

    lane = lax.iota(jnp.int32, L)

    def body(j, carry):
        bk0, bv0, bk1, bv1 = carry
        ck0 = dbuf[pl.ds(j * L, L)]
        ck1 = dbuf[pl.ds(HCH + j * L, L)]
        cv0 = gbase + base + j * L + lane
        cv1 = gbase + base + HCH + j * L + lane
        ck0s, cv0s = plsc.sort_key_val(ck0, cv0)
        ck1s, cv1s = plsc.sort_key_val(ck1, cv1)
        bk0, bv0 = _merge16(bk0, bv0, ck0s, cv0s)
        bk1, bv1 = _merge16(bk1, bv1, ck1s, cv1s)
        return (bk0, bv0, bk1, bv1)

    init = (jnp.full((L,), jnp.inf, jnp.float32),
            jnp.zeros((L,), jnp.int32),
            jnp.full((L,), jnp.inf, jnp.float32),
            jnp.zeros((L,), jnp.int32))
    bk0, bv0, bk1, bv1 = lax.fori_loop(0, HCH // L, body, init)
    return _merge16(bk0, bv0, bk1, bv1)


@functools.cache
def _sc_kernels():
    mesh = plsc.VectorSubcoreMesh(core_axis_name="c", subcore_axis_name="s",
                                  num_cores=NC, num_subcores=NS)

    @functools.partial(
        pl.kernel,
        out_type=(
            jax.ShapeDtypeStruct((NC, NS, L), jnp.float32),
            jax.ShapeDtypeStruct((NC, NS, L), jnp.int32),
        ),
        mesh=mesh,
        scratch_types=[
            pltpu.VMEM((CH,), jnp.float32),
            pltpu.VMEM((L,), jnp.float32),
            pltpu.VMEM((L,), jnp.int32),
        ],
        compiler_params=pltpu.CompilerParams(needs_layout_passes=False),
    )
    def _local_topk(dist_hbm, outk_hbm, outi_hbm, dbuf, kbuf, ibuf):
        c = lax.axis_index("c")
        s = lax.axis_index("s")
        bk, bv = _scan_chunk(dist_hbm, dbuf, s, 0)
        kbuf[...] = bk
        ibuf[...] = bv
        pltpu.sync_copy(kbuf, outk_hbm.at[c, s])
        pltpu.sync_copy(ibuf, outi_hbm.at[c, s])

    @functools.partial(
        pl.kernel,
        out_type=(
            jax.ShapeDtypeStruct((L,), jnp.int32),
            jax.ShapeDtypeStruct((NC, NS, L), jnp.float32),
            jax.ShapeDtypeStruct((NC, NS, L), jnp.int32),
        ),
        mesh=mesh,
        scratch_types=[
            pltpu.VMEM((CH,), jnp.float32),
            pltpu.VMEM((L,), jnp.float32),
            pltpu.VMEM((L,), jnp.int32),
            pltpu.VMEM((NS, L), jnp.float32),
            pltpu.VMEM((NS, L), jnp.int32),
            pltpu.VMEM((NS, L), jnp.float32),
            pltpu.VMEM((NS, L), jnp.int32),
            pltpu.VMEM((L,), jnp.int32),
            pltpu.VMEM((L,), jnp.int32),
            pltpu.VMEM((NREF, D), jnp.float32),
            pltpu.VMEM((D,), jnp.float32),
            pltpu.VMEM((L,), jnp.int32),
            pltpu.SemaphoreType.DMA,
        ],
        compiler_params=pltpu.CompilerParams(needs_layout_passes=False),
    )
    def _final_predict(dist_hbm, candk_hbm, candi_hbm, x_hbm, train_hbm,
                       labels_hbm, out_hbm, xk_hbm, xi_hbm,
                       dbuf, kbuf, ibuf, kb, ib, kb2, ib2,
                       idx0, idx1, rows, xb, lanebuf, sem):
        c = lax.axis_index("c")
        s = lax.axis_index("s")
        bk, bv = _scan_chunk(dist_hbm, dbuf, s, NH)
        kbuf[...] = bk
        ibuf[...] = bv
        pltpu.sync_copy(kbuf, xk_hbm.at[c, s])
        pltpu.sync_copy(ibuf, xi_hbm.at[c, s])
        pltpu.sync_copy(x_hbm, xb)
        pltpu.sync_copy(candk_hbm.at[c], kb2)
        pltpu.sync_copy(candi_hbm.at[c], ib2)
        plsc.subcore_barrier()
        pltpu.sync_copy(xk_hbm.at[c], kb)
        pltpu.sync_copy(xi_hbm.at[c], ib)
        b0k = kb[0]
        b0v = ib[0]
        b1k = jnp.full((L,), jnp.inf, jnp.float32)
        b1v = jnp.zeros((L,), jnp.int32)
        for j in range(1, 2 * NS):
            rk = kb[j] if j < NS else kb2[j - NS]
            rv = ib[j] if j < NS else ib2[j - NS]
            b0k, b0v, uk, uv = _merge16_both(b0k, b0v, rk, rv)
            b1k, b1v = _merge16(b1k, b1v, uk, uv)
        idx0[...] = b0v
        idx1[...] = b1v
        pltpu.async_copy(train_hbm.at[idx0], rows.at[0:L], sem).wait()
        pltpu.async_copy(train_hbm.at[idx1], rows.at[L:NREF], sem).wait()

        def exact_d2(r):
            acc = jnp.zeros((L,), jnp.float32)
            for h in range(D // L):
                tv = rows[r, pl.ds(h * L, L)]
                xv = xb[pl.ds(h * L, L)]
                dv = tv - xv
                acc = acc + dv * dv
            return jnp.full((L,), jnp.sum(acc, axis=0), jnp.float32)

        lane = lax.iota(jnp.int32, L)
        e0k = jnp.zeros((L,), jnp.float32)
        e1k = jnp.zeros((L,), jnp.float32)
        for r in range(L):
            e0k = jnp.where(lane == r, exact_d2(r), e0k)
            e1k = jnp.where(lane == r, exact_d2(L + r), e1k)
        e0k, e0v = plsc.sort_key_val(e0k, b0v)
        e1k, e1v = plsc.sort_key_val(e1k, b1v)
        fk, fv = _merge16(e0k, e0v, e1k, e1v)
        idx0[...] = fv
        pltpu.async_copy(labels_hbm.at[idx0], lanebuf, sem).wait()
        lv = lanebuf[...]
        valid = lane < K
        best_cnt = jnp.full((L,), -1, jnp.int32)
        best_cls = jnp.zeros((L,), jnp.int32)
        for cls in range(NUM_CLASSES):
            mm = jnp.logical_and(lv == cls, valid)
            cnt = plsc.all_reduce_population_count(mm)
            upd = cnt > best_cnt
            best_cnt = jnp.where(upd, cnt, best_cnt)
            best_cls = jnp.where(
                upd, jnp.full((L,), cls, jnp.int32), best_cls)
        idx1[...] = best_cls

        @pl.when(jnp.logical_and(c == 0, s == 0))
        def _():
            pltpu.sync_copy(idx1, out_hbm)

    return _local_topk, _final_predict


def kernel(x, train_data, train_labels):
    local_topk, final_predict = _sc_kernels()
    x2 = x.reshape(1, D)
    dist0 = _dist_half0(x2, train_data).reshape(NH)
    candk, candi = local_topk(dist0)
    dist1 = _dist_half1(x2, train_data).reshape(NH)
    pred, _, _ = final_predict(dist1, candk, candi, x, train_data,
                               train_labels)
    return pred[0]


# speedup vs baseline: 1.0644x; 1.0644x over previous
import functools

import jax
import jax.numpy as jnp
from jax import lax
from jax.experimental import pallas as pl
from jax.experimental.pallas import tpu as pltpu
from jax.experimental.pallas import tpu_sc as plsc

N = 100000
D = 128
K = 15
NUM_CLASSES = 10

NB = 20
B = N // NB

NC = 2
NS = 16
L = 16
CP = 6272
CSHORT = N - (NS - 1) * CP
HCP = CP // 2
NREF = 32


def _dist_body(x_ref, t_ref, o_ref):
    t = t_ref[...]
    diff = t - x_ref[...]
    sq = (diff * diff).astype(jnp.bfloat16)
    ones = jnp.ones((1, D), jnp.bfloat16)
    d = lax.dot_general(ones, sq, (((1,), (1,)), ((), ())),
                        preferred_element_type=jnp.float32)
    o_ref[...] = d.reshape(1, 1, B)


_dist_call = pl.pallas_call(
    _dist_body,
    grid=(NB,),
    in_specs=[
        pl.BlockSpec((1, D), lambda i: (0, 0)),
        pl.BlockSpec((B, D), lambda i: (i, 0)),
    ],
    out_specs=pl.BlockSpec((1, 1, B), lambda i: (i, 0, 0)),
    out_shape=jax.ShapeDtypeStruct((NB, 1, B), jnp.float32),
)


def _merge16(bk, bv, ck, cv):
    ckr = lax.rev(ck, (0,))
    cvr = lax.rev(cv, (0,))
    m = ckr < bk
    mk = jnp.where(m, ckr, bk)
    mv = jnp.where(m, cvr, bv)
    mk2, mv2 = plsc.sort_key_val(mk, mv)
    return mk2, mv2


def _merge16_both(bk, bv, ck, cv):
    ckr = lax.rev(ck, (0,))
    cvr = lax.rev(cv, (0,))
    m = ckr < bk
    lk = jnp.where(m, ckr, bk)
    lv = jnp.where(m, cvr, bv)
    uk = jnp.where(m, bk, ckr)
    uv = jnp.where(m, bv, cvr)
    lk2, lv2 = plsc.sort_key_val(lk, lv)
    uk2, uv2 = plsc.sort_key_val(uk, uv)
    return lk2, lv2, uk2, uv2


@functools.cache
def _sc_kernel():
    mesh = plsc.VectorSubcoreMesh(core_axis_name="c", subcore_axis_name="s",
                                  num_cores=NC, num_subcores=NS)

    @functools.partial(
        pl.kernel,
        out_type=(
            jax.ShapeDtypeStruct((L,), jnp.int32),
            jax.ShapeDtypeStruct((NC, NS, L), jnp.float32),
            jax.ShapeDtypeStruct((NC, NS, L), jnp.int32),
        ),
        mesh=mesh,
        scratch_types=[
            pltpu.VMEM((CP,), jnp.float32),
            pltpu.VMEM((L,), jnp.float32),
            pltpu.VMEM((L,), jnp.int32),
            pltpu.VMEM((NS, L), jnp.float32),
            pltpu.VMEM((NS, L), jnp.int32),
            pltpu.VMEM((L,), jnp.int32),
            pltpu.VMEM((L,), jnp.int32),
            pltpu.VMEM((NREF, D), jnp.float32),
            pltpu.VMEM((D,), jnp.float32),
            pltpu.VMEM((L,), jnp.int32),
            pltpu.SemaphoreType.DMA,
        ],
        compiler_params=pltpu.CompilerParams(needs_layout_passes=False),
    )
    def _topk_predict(dist_hbm, x_hbm, train_hbm, labels_hbm,
                      out_hbm, xk_hbm, xi_hbm,
                      dbuf, kbuf, ibuf, kb, ib,
                      idx0, idx1, rows, xb, lanebuf, sem):
        c = lax.axis_index("c")
        s = lax.axis_index("s")
        base = s * CP
        inf16 = jnp.full((L,), jnp.inf, jnp.float32)
        for t in range((CP - CSHORT) // L):
            dbuf[pl.ds(CSHORT + t * L, L)] = inf16

        @pl.when(s < NS - 1)
        def _():
            pltpu.sync_copy(dist_hbm.at[pl.ds(base, CP)], dbuf)

        @pl.when(s == NS - 1)
        def _():
            pltpu.sync_copy(dist_hbm.at[pl.ds(base, CSHORT)],
                            dbuf.at[pl.ds(0, CSHORT)])

        pltpu.sync_copy(x_hbm, xb)
        lane = lax.iota(jnp.int32, L)

        def body(j, carry):
            bk0, bv0, bk1, bv1 = carry
            ck0 = dbuf[pl.ds(j * L, L)]
            ck1 = dbuf[pl.ds(HCP + j * L, L)]
            cv0 = base + j * L + lane
            cv1 = base + HCP + j * L + lane
            ck0s, cv0s = plsc.sort_key_val(ck0, cv0)
            ck1s, cv1s = plsc.sort_key_val(ck1, cv1)
            bk0, bv0 = _merge16(bk0, bv0, ck0s, cv0s)
            bk1, bv1 = _merge16(bk1, bv1, ck1s, cv1s)
            return (bk0, bv0, bk1, bv1)

        init = (jnp.full((L,), jnp.inf, jnp.float32),
                jnp.zeros((L,), jnp.int32),
                jnp.full((L,), jnp.inf, jnp.float32),
                jnp.zeros((L,), jnp.int32))
        bk0, bv0, bk1, bv1 = lax.fori_loop(0, HCP // L, body, init)
        bk, bv = _merge16(bk0, bv0, bk1, bv1)
        kbuf[...] = bk
        ibuf[...] = bv
        pltpu.sync_copy(kbuf, xk_hbm.at[c, s])
        pltpu.sync_copy(ibuf, xi_hbm.at[c, s])
        plsc.subcore_barrier()
        pltpu.sync_copy(xk_hbm.at[c], kb)
        pltpu.sync_copy(xi_hbm.at[c], ib)
        b0k = kb[0]
        b0v = ib[0]
        b1k = jnp.full((L,), jnp.inf, jnp.float32)
        b1v = jnp.zeros((L,), jnp.int32)
        for j in range(1, NS):
            b0k, b0v, uk, uv = _merge16_both(b0k, b0v, kb[j], ib[j])
            b1k, b1v = _merge16(b1k, b1v, uk, uv)
        idx0[...] = b0v
        idx1[...] = b1v
        pltpu.async_copy(train_hbm.at[idx0], rows.at[0:L], sem).wait()
        pltpu.async_copy(train_hbm.at[idx1], rows.at[L:NREF], sem).wait()

        def exact_d2(r):
            acc = jnp.zeros((L,), jnp.float32)
            for h in range(D // L):
                tv = rows[r, pl.ds(h * L, L)]
                xv = xb[pl.ds(h * L, L)]
                dv = tv - xv
                acc = acc + dv * dv
            return jnp.full((L,), jnp.sum(acc, axis=0), jnp.float32)

        e0k = jnp.zeros((L,), jnp.float32)
        e1k = jnp.zeros((L,), jnp.float32)
        for r in range(L):
            e0k = jnp.where(lane == r, exact_d2(r), e0k)
            e1k = jnp.where(lane == r, exact_d2(L + r), e1k)
        e0k, e0v = plsc.sort_key_val(e0k, b0v)
        e1k, e1v = plsc.sort_key_val(e1k, b1v)
        fk, fv = _merge16(e0k, e0v, e1k, e1v)
        idx0[...] = fv
        pltpu.async_copy(labels_hbm.at[idx0], lanebuf, sem).wait()
        lv = lanebuf[...]
        valid = lane < K
        best_cnt = jnp.full((L,), -1, jnp.int32)
        best_cls = jnp.zeros((L,), jnp.int32)
        for cls in range(NUM_CLASSES):
            mm = jnp.logical_and(lv == cls, valid)
            cnt = plsc.all_reduce_population_count(mm)
            upd = cnt > best_cnt
            best_cnt = jnp.where(upd, cnt, best_cnt)
            best_cls = jnp.where(
                upd, jnp.full((L,), cls, jnp.int32), best_cls)
        idx1[...] = best_cls

        @pl.when(jnp.logical_and(c == 0, s == 0))
        def _():
            pltpu.sync_copy(idx1, out_hbm)

    return _topk_predict


def kernel(x, train_data, train_labels):
    topk_predict = _sc_kernel()
    dist = _dist_call(x.reshape(1, D), train_data).reshape(N)
    pred, _, _ = topk_predict(dist, x, train_data, train_labels)
    return pred[0]
